# baseline (speedup 1.0000x reference)
import functools

import jax
import jax.numpy as jnp
from jax import lax
from jax.experimental import pallas as pl
from jax.experimental.pallas import tpu as pltpu
from jax.experimental.pallas import tpu_sc as plsc

_VOCAB = 113
_VPAD = 128
_D = 512
_L = 200
_NC = 2
_NS = 16
_NW = _NC * _NS
_LANES = 16
_W = 64
_NBUF = 2


def _pos_encoding(max_length, d_model):
    even_i = jnp.arange(0, d_model, 2).astype(jnp.float32)
    denominator = jnp.power(jnp.float32(10000.0), even_i / d_model)
    position = jnp.arange(max_length, dtype=jnp.float32).reshape(max_length, 1)
    even_pe = jnp.sin(position / denominator)
    odd_pe = jnp.cos(position / denominator)
    return jnp.stack([even_pe, odd_pe], axis=2).reshape(max_length, d_model)


def _ctable_body(table_ref, pe_ref, out_ref):
    out_ref[...] = table_ref[...][None, :, :] + pe_ref[...][:, None, :]


@jax.jit
def _build_ctable(table_pad, pe):
    t_blk = 8
    return pl.pallas_call(
        _ctable_body,
        grid=(_L // t_blk,),
        in_specs=[
            pl.BlockSpec((_VPAD, _D), lambda i: (0, 0)),
            pl.BlockSpec((t_blk, _D), lambda i: (i, 0)),
        ],
        out_specs=pl.BlockSpec((t_blk, _VPAD, _D), lambda i: (i, 0, 0)),
        out_shape=jax.ShapeDtypeStruct((_L, _VPAD, _D), jnp.float32),
    )(table_pad, pe)


def _make_sc_gather(n_tokens):
    per_w = n_tokens // _NW
    n_chunks = per_w // _W
    n_groups = n_chunks // _NBUF
    mesh = plsc.VectorSubcoreMesh(core_axis_name="c", subcore_axis_name="s")

    @functools.partial(
        pl.kernel, mesh=mesh,
        out_type=jax.ShapeDtypeStruct((n_tokens, _D), jnp.float32),
        scratch_types=[
            pltpu.VMEM((_NBUF, _W), jnp.int32),
            pltpu.VMEM((_NBUF, _W, _D), jnp.float32),
            pltpu.SemaphoreType.DMA((_NBUF,)),
            pltpu.SemaphoreType.DMA((_NBUF,)),
        ],
    )
    def sc_gather(ctable_hbm, x_hbm, out_hbm, idx_v, rows_v, gsem, wsem):
        wid = lax.axis_index("s") * _NC + lax.axis_index("c")
        base = wid * per_w

        def load_and_gather(c, b):
            off = base + c * _W
            pltpu.sync_copy(x_hbm.at[pl.ds(off, _W)], idx_v.at[b])

            @pl.loop(0, _W // _LANES)
            def _(k):
                lane_n = off + k * _LANES + lax.broadcasted_iota(
                    jnp.int32, (_LANES,), 0)
                t = lax.rem(lane_n, _L)
                sl = pl.ds(k * _LANES, _LANES)
                idx_v[b, sl] = idx_v[b, sl] + t * _VPAD

            pltpu.async_copy(ctable_hbm.at[idx_v.at[b]], rows_v.at[b],
                             gsem.at[b])

        for b in range(_NBUF):
            load_and_gather(b, b)

        @pl.loop(0, n_groups)
        def _(g):
            writes = []
            for b in range(_NBUF):
                c = g * _NBUF + b
                off = base + c * _W
                pltpu.make_async_copy(out_hbm.at[pl.ds(off, _W)],
                                      rows_v.at[b], gsem.at[b]).wait()
                writes.append(pltpu.async_copy(
                    rows_v.at[b], out_hbm.at[pl.ds(off, _W)], wsem.at[b]))
            for b in range(_NBUF):
                c2 = g * _NBUF + b + _NBUF
                writes[b].wait()

                @pl.when(c2 < n_chunks)
                def _():
                    load_and_gather(c2, b)

    return sc_gather


def _tc_body(x_ref, table_ref, pe_ref, out_ref):
    n = x_ref.shape[0]
    onehot = (x_ref[...] ==
              jax.lax.broadcasted_iota(jnp.int32, (n, _VPAD), 1))
    emb = jnp.dot(onehot.astype(jnp.bfloat16), table_ref[...],
                  preferred_element_type=jnp.float32)
    out_ref[...] = emb + pe_ref[...]


def _tc_lookup(x2, table_pad_bf16, pe_tiled):
    n_tokens = x2.shape[0]
    rows = pe_tiled.shape[0]
    return pl.pallas_call(
        _tc_body,
        grid=(n_tokens // rows,),
        in_specs=[
            pl.BlockSpec((rows, 1), lambda i: (i, 0)),
            pl.BlockSpec((_VPAD, _D), lambda i: (0, 0)),
            pl.BlockSpec((rows, _D), lambda i: (0, 0)),
        ],
        out_specs=pl.BlockSpec((rows, _D), lambda i: (i, 0)),
        out_shape=jax.ShapeDtypeStruct((n_tokens, _D), jnp.float32),
    )(x2, table_pad_bf16, pe_tiled)


_S_SC = 512


@jax.jit
def _run(x_flat, table_pad, table_pad_bf16, pe, pe_tiled):
    t_sc = _S_SC * _L
    ctable = _build_ctable(table_pad, pe).reshape(_L * _VPAD, _D)
    out_sc = _make_sc_gather(t_sc)(ctable, x_flat[:t_sc])
    out_tc = _tc_lookup(x_flat[t_sc:].reshape(-1, 1), table_pad_bf16,
                        pe_tiled)
    return jnp.concatenate([out_sc, out_tc], axis=0)


def kernel(x, table):
    batch, length = x.shape
    pe = _pos_encoding(_L, _D)
    pe_tiled = jnp.tile(pe, (8, 1))
    table_pad = jnp.zeros((_VPAD, _D), jnp.float32).at[:_VOCAB].set(table)
    x_flat = x.astype(jnp.int32).reshape(batch * length)
    out = _run(x_flat, table_pad, table_pad.astype(jnp.bfloat16), pe,
               pe_tiled)
    return out.reshape(batch, length, _D)


# speedup vs baseline: 2.5939x; 2.5939x over previous
import functools

import jax
import jax.numpy as jnp
from jax import lax
from jax.experimental import pallas as pl
from jax.experimental.pallas import tpu as pltpu
from jax.experimental.pallas import tpu_sc as plsc

_VOCAB = 113
_VPAD = 128
_D = 512
_L = 200
_NC = 2
_NS = 16
_NW = _NC * _NS
_LANES = 16
_W = 64
_NBUF = 2


def _pos_encoding(max_length, d_model):
    even_i = jnp.arange(0, d_model, 2).astype(jnp.float32)
    denominator = jnp.power(jnp.float32(10000.0), even_i / d_model)
    position = jnp.arange(max_length, dtype=jnp.float32).reshape(max_length, 1)
    even_pe = jnp.sin(position / denominator)
    odd_pe = jnp.cos(position / denominator)
    return jnp.stack([even_pe, odd_pe], axis=2).reshape(max_length, d_model)


def _ctable_body(table_ref, pe_ref, out_ref):
    out_ref[...] = table_ref[...][None, :, :] + pe_ref[...][:, None, :]


@jax.jit
def _build_ctable(table_pad, pe):
    t_blk = 8
    return pl.pallas_call(
        _ctable_body,
        grid=(_L // t_blk,),
        in_specs=[
            pl.BlockSpec((_VPAD, _D), lambda i: (0, 0)),
            pl.BlockSpec((t_blk, _D), lambda i: (i, 0)),
        ],
        out_specs=pl.BlockSpec((t_blk, _VPAD, _D), lambda i: (i, 0, 0)),
        out_shape=jax.ShapeDtypeStruct((_L, _VPAD, _D), jnp.float32),
    )(table_pad, pe)


def _make_sc_gather(n_tokens):
    per_w = n_tokens // _NW
    n_chunks = per_w // _W
    n_groups = n_chunks // _NBUF
    mesh = plsc.VectorSubcoreMesh(core_axis_name="c", subcore_axis_name="s")

    @functools.partial(
        pl.kernel, mesh=mesh,
        out_type=jax.ShapeDtypeStruct((n_tokens, _D), jnp.float32),
        scratch_types=[
            pltpu.VMEM((_NBUF, _W), jnp.int32),
            pltpu.VMEM((_NBUF, _W, _D), jnp.float32),
            pltpu.SemaphoreType.DMA((_NBUF,)),
            pltpu.SemaphoreType.DMA((_NBUF,)),
        ],
    )
    def sc_gather(ctable_hbm, x_hbm, out_hbm, idx_v, rows_v, gsem, wsem):
        wid = lax.axis_index("s") * _NC + lax.axis_index("c")
        base = wid * per_w

        def load_and_gather(c, b):
            off = base + c * _W
            pltpu.sync_copy(x_hbm.at[pl.ds(off, _W)], idx_v.at[b])

            @pl.loop(0, _W // _LANES)
            def _(k):
                lane_n = off + k * _LANES + lax.broadcasted_iota(
                    jnp.int32, (_LANES,), 0)
                t = lax.rem(lane_n, _L)
                sl = pl.ds(k * _LANES, _LANES)
                idx_v[b, sl] = idx_v[b, sl] + t * _VPAD

            pltpu.async_copy(ctable_hbm.at[idx_v.at[b]], rows_v.at[b],
                             gsem.at[b])

        for b in range(_NBUF):
            load_and_gather(b, b)

        @pl.loop(0, n_groups)
        def _(g):
            writes = []
            for b in range(_NBUF):
                c = g * _NBUF + b
                off = base + c * _W
                pltpu.make_async_copy(out_hbm.at[pl.ds(off, _W)],
                                      rows_v.at[b], gsem.at[b]).wait()
                writes.append(pltpu.async_copy(
                    rows_v.at[b], out_hbm.at[pl.ds(off, _W)], wsem.at[b]))
            for b in range(_NBUF):
                c2 = g * _NBUF + b + _NBUF
                writes[b].wait()

                @pl.when(c2 < n_chunks)
                def _():
                    load_and_gather(c2, b)

    return sc_gather


def _tc_body(x_ref, table_ref, pe_ref, out_ref):
    n = x_ref.shape[0]
    onehot = (x_ref[...] ==
              jax.lax.broadcasted_iota(jnp.int32, (n, _VPAD), 1))
    emb = jnp.dot(onehot.astype(jnp.bfloat16), table_ref[...],
                  preferred_element_type=jnp.float32)
    out_ref[...] = emb + pe_ref[...]


def _tc_lookup(x2, table_pad_bf16, pe_tiled):
    n_tokens = x2.shape[0]
    rows = pe_tiled.shape[0]
    return pl.pallas_call(
        _tc_body,
        grid=(n_tokens // rows,),
        in_specs=[
            pl.BlockSpec((rows, 1), lambda i: (i, 0)),
            pl.BlockSpec((_VPAD, _D), lambda i: (0, 0)),
            pl.BlockSpec((rows, _D), lambda i: (0, 0)),
        ],
        out_specs=pl.BlockSpec((rows, _D), lambda i: (i, 0)),
        out_shape=jax.ShapeDtypeStruct((n_tokens, _D), jnp.float32),
    )(x2, table_pad_bf16, pe_tiled)


_S_SC = 0


@jax.jit
def _run(x_flat, table_pad, table_pad_bf16, pe, pe_tiled):
    if _S_SC == 0:
        return _tc_lookup(x_flat.reshape(-1, 1), table_pad_bf16, pe_tiled)
    t_sc = _S_SC * _L
    ctable = _build_ctable(table_pad, pe).reshape(_L * _VPAD, _D)
    out_sc = _make_sc_gather(t_sc)(ctable, x_flat[:t_sc])
    out_tc = _tc_lookup(x_flat[t_sc:].reshape(-1, 1), table_pad_bf16,
                        pe_tiled)
    return jnp.concatenate([out_sc, out_tc], axis=0)


def kernel(x, table):
    batch, length = x.shape
    pe = _pos_encoding(_L, _D)
    pe_tiled = jnp.tile(pe, (32, 1))
    table_pad = jnp.zeros((_VPAD, _D), jnp.float32).at[:_VOCAB].set(table)
    x_flat = x.astype(jnp.int32).reshape(batch * length)
    out = _run(x_flat, table_pad, table_pad.astype(jnp.bfloat16), pe,
               pe_tiled)
    return out.reshape(batch, length, _D)


# speedup vs baseline: 2.7136x; 1.0461x over previous
import functools

import jax
import jax.numpy as jnp
from jax import lax
from jax.experimental import pallas as pl
from jax.experimental.pallas import tpu as pltpu
from jax.experimental.pallas import tpu_sc as plsc

_VOCAB = 113
_VPAD = 128
_D = 512
_L = 200
_NC = 2
_NS = 16
_NW = _NC * _NS
_LANES = 16
_W = 64
_NBUF = 2


def _pos_encoding(max_length, d_model):
    even_i = jnp.arange(0, d_model, 2).astype(jnp.float32)
    denominator = jnp.power(jnp.float32(10000.0), even_i / d_model)
    position = jnp.arange(max_length, dtype=jnp.float32).reshape(max_length, 1)
    even_pe = jnp.sin(position / denominator)
    odd_pe = jnp.cos(position / denominator)
    return jnp.stack([even_pe, odd_pe], axis=2).reshape(max_length, d_model)


def _ctable_body(table_ref, pe_ref, out_ref):
    out_ref[...] = table_ref[...][None, :, :] + pe_ref[...][:, None, :]


@jax.jit
def _build_ctable(table_pad, pe):
    t_blk = 8
    return pl.pallas_call(
        _ctable_body,
        grid=(_L // t_blk,),
        in_specs=[
            pl.BlockSpec((_VPAD, _D), lambda i: (0, 0)),
            pl.BlockSpec((t_blk, _D), lambda i: (i, 0)),
        ],
        out_specs=pl.BlockSpec((t_blk, _VPAD, _D), lambda i: (i, 0, 0)),
        out_shape=jax.ShapeDtypeStruct((_L, _VPAD, _D), jnp.float32),
    )(table_pad, pe)


def _make_sc_gather(n_tokens):
    per_w = n_tokens // _NW
    n_chunks = per_w // _W
    n_groups = n_chunks // _NBUF
    mesh = plsc.VectorSubcoreMesh(core_axis_name="c", subcore_axis_name="s")

    @functools.partial(
        pl.kernel, mesh=mesh,
        out_type=jax.ShapeDtypeStruct((n_tokens, _D), jnp.float32),
        scratch_types=[
            pltpu.VMEM((_NBUF, _W), jnp.int32),
            pltpu.VMEM((_NBUF, _W, _D), jnp.float32),
            pltpu.SemaphoreType.DMA((_NBUF,)),
            pltpu.SemaphoreType.DMA((_NBUF,)),
        ],
    )
    def sc_gather(ctable_hbm, x_hbm, out_hbm, idx_v, rows_v, gsem, wsem):
        wid = lax.axis_index("s") * _NC + lax.axis_index("c")
        base = wid * per_w

        def load_and_gather(c, b):
            off = base + c * _W
            pltpu.sync_copy(x_hbm.at[pl.ds(off, _W)], idx_v.at[b])

            @pl.loop(0, _W // _LANES)
            def _(k):
                lane_n = off + k * _LANES + lax.broadcasted_iota(
                    jnp.int32, (_LANES,), 0)
                t = lax.rem(lane_n, _L)
                sl = pl.ds(k * _LANES, _LANES)
                idx_v[b, sl] = idx_v[b, sl] + t * _VPAD

            pltpu.async_copy(ctable_hbm.at[idx_v.at[b]], rows_v.at[b],
                             gsem.at[b])

        for b in range(_NBUF):
            load_and_gather(b, b)

        @pl.loop(0, n_groups)
        def _(g):
            writes = []
            for b in range(_NBUF):
                c = g * _NBUF + b
                off = base + c * _W
                pltpu.make_async_copy(out_hbm.at[pl.ds(off, _W)],
                                      rows_v.at[b], gsem.at[b]).wait()
                writes.append(pltpu.async_copy(
                    rows_v.at[b], out_hbm.at[pl.ds(off, _W)], wsem.at[b]))
            for b in range(_NBUF):
                c2 = g * _NBUF + b + _NBUF
                writes[b].wait()

                @pl.when(c2 < n_chunks)
                def _():
                    load_and_gather(c2, b)

    return sc_gather


def _tc_body(x_ref, table_ref, pe_ref, out_ref):
    n = x_ref.shape[0]
    b = out_ref.shape[0]
    onehot = (x_ref[...] ==
              jax.lax.broadcasted_iota(jnp.int32, (n, _VPAD), 1))
    emb = jnp.dot(onehot.astype(jnp.bfloat16), table_ref[...],
                  preferred_element_type=jnp.float32)
    out_ref[...] = emb.reshape(b, _L, _D) + pe_ref[...][None, :, :]


def _tc_lookup(x2, table_pad_bf16, pe, block_b):
    n_tokens = x2.shape[0]
    rows = block_b * _L
    return pl.pallas_call(
        _tc_body,
        grid=(n_tokens // rows,),
        in_specs=[
            pl.BlockSpec((rows, 1), lambda i: (i, 0)),
            pl.BlockSpec((_VPAD, _D), lambda i: (0, 0)),
            pl.BlockSpec((_L, _D), lambda i: (0, 0)),
        ],
        out_specs=pl.BlockSpec((block_b, _L, _D), lambda i: (i, 0, 0)),
        out_shape=jax.ShapeDtypeStruct((n_tokens // _L, _L, _D), jnp.float32),
    )(x2, table_pad_bf16, pe)


_S_SC = 0


_TC_BLOCK_B = 32


@jax.jit
def _run(x_flat, table_pad, table_pad_bf16, pe):
    if _S_SC == 0:
        out = _tc_lookup(x_flat.reshape(-1, 1), table_pad_bf16, pe,
                         _TC_BLOCK_B)
        return out.reshape(-1, _D)
    t_sc = _S_SC * _L
    ctable = _build_ctable(table_pad, pe).reshape(_L * _VPAD, _D)
    out_sc = _make_sc_gather(t_sc)(ctable, x_flat[:t_sc])
    out_tc = _tc_lookup(x_flat[t_sc:].reshape(-1, 1), table_pad_bf16,
                        pe, _TC_BLOCK_B).reshape(-1, _D)
    return jnp.concatenate([out_sc, out_tc], axis=0)


def kernel(x, table):
    batch, length = x.shape
    pe = _pos_encoding(_L, _D)
    table_pad = jnp.zeros((_VPAD, _D), jnp.float32).at[:_VOCAB].set(table)
    x_flat = x.astype(jnp.int32).reshape(batch * length)
    out = _run(x_flat, table_pad, table_pad.astype(jnp.bfloat16), pe)
    return out.reshape(batch, length, _D)
